# Initial kernel scaffold; baseline (speedup 1.0000x reference)
import jax
import jax.numpy as jnp
from jax.experimental import pallas as pl


def kernel(input, target):
    raise NotImplementedError("write your pallas kernel here")


# speedup vs baseline: 32.1953x; 32.1953x over previous
import functools

import numpy as np
import jax
import jax.numpy as jnp
from jax import lax
from jax.experimental import pallas as pl
from jax.experimental.pallas import tpu as pltpu
from jax.experimental.pallas import tpu_sc as plsc

_B = 1024
_Y = 1000
_MAXT = 64
_NSTREAM = 80 * 1024
_RCHUNK = 16
_NCHUNK = _B // _RCHUNK

_STREAM_HOST = (
    np.random.RandomState(0)
    .randint(0, 2**32, size=_NSTREAM, dtype=np.uint32)
    .view(np.int32)
)
_LTAB_HOST = np.log(
    np.floor((float(_Y) - 1.0) / np.arange(1, _MAXT + 1, dtype=np.float64))
).astype(np.float32)


def _max_run(bad):
    idx = np.flatnonzero(np.diff(np.concatenate(([0], bad.view(np.uint8), [0]))))
    return int((idx[1::2] - idx[::2]).max()) if idx.size else 0


_SU = _STREAM_HOST.view(np.uint32)
_NEG_DEPTH = 1 + _max_run((_SU & 1023) > 935)
_POS_DEPTH = 1 + max(_max_run((_SU & 3) == 3), _max_run((_SU & 7) > 4))
assert _NEG_DEPTH <= 16 and _POS_DEPTH <= 16


def _extract_body(t_ref, meta_ref):
    t = t_ref[...]
    iota = lax.broadcasted_iota(jnp.int32, (_B, _Y), 1)
    big = jnp.int32(2048)
    live = jnp.where(t > 0.0, iota, big)
    cols = []
    for _ in range(5):
        mn = jnp.min(live, axis=1)
        cols.append(mn)
        live = jnp.where(live == mn[:, None], big, live)
    npos = jnp.sum((t > 0.0).astype(jnp.int32), axis=1)
    zero = jnp.zeros((_B,), jnp.int32)
    meta_ref[...] = jnp.stack(cols + [npos, zero, zero], axis=1)


_extract = pl.pallas_call(
    _extract_body,
    out_shape=jax.ShapeDtypeStruct((_B, 8), jnp.int32),
)


def _sc_body(inp_hbm, meta_hbm, stream_hbm, ltab_hbm, out_hbm,
             stream_v, meta_v, rows_v, ltab_v, out_v, dset_s, sem0, sem1):
    cid = lax.axis_index("c")
    sid = lax.axis_index("s")

    @pl.when(jnp.logical_and(cid == 0, sid == 0))
    def _serial():
        pltpu.sync_copy(stream_hbm, stream_v.at[pl.ds(0, _NSTREAM)])
        pltpu.sync_copy(meta_hbm, meta_v.at[pl.ds(0, _B * 8)])
        pltpu.sync_copy(ltab_hbm, ltab_v.at[pl.ds(0, _MAXT)])

        csz = _RCHUNK * _Y

        def chunk_copy(c, buf, sem):
            return pltpu.make_async_copy(
                inp_hbm.at[pl.ds(c * csz, csz)],
                rows_v.at[buf, pl.ds(0, csz)], sem)

        chunk_copy(0, 0, sem0).start()
        chunk_copy(1, 1, sem1).start()

        def draw(p, rngv, mask, depth):
            def attempt(k):
                w = stream_v[pl.ds(p + k, 16)][0]
                v = jnp.bitwise_and(w, mask)
                if k == depth - 1:
                    return (v, p + jnp.int32(k + 1))
                return lax.cond(v <= rngv,
                                lambda: (v, p + jnp.int32(k + 1)),
                                lambda: attempt(k + 1))

            return attempt(0)

        def make_row_body(buf):
            def row_body(r, carry, c):
                p, acc = carry
                i = c * _RCHUNK + r
                roff = r * _Y
                npos = meta_v[pl.ds(i * 8 + 5, 16)][0]

                def posdraw(pp):
                    rngv = npos - 1
                    m = rngv
                    m = lax.bitwise_or(m, m >> 1)
                    m = lax.bitwise_or(m, m >> 2)
                    return draw(pp, rngv, m, _POS_DEPTH)

                rr, p = lax.cond(npos > 1, posdraw,
                                 lambda pp: (jnp.int32(0), pp), p)
                j = meta_v[pl.ds(i * 8 + rr, 16)][0]
                xj = rows_v[buf, pl.ds(roff + j, 16)][0]
                dset_s[0] = j

                def passive(st):
                    return st

                def make_active(k):
                    def active(st):
                        _, _, p = st
                        rngv2 = jnp.int32(_Y - 2) - k
                        r2, p = draw(p, rngv2, jnp.int32(1023), _NEG_DEPTH)

                        def fp_body(_, s):
                            m_prev, m = s

                            def fp_step(s2):
                                _, m2 = s2

                                def cbody(q, cnt):
                                    return cnt + jnp.where(
                                        dset_s[q] <= m2,
                                        jnp.int32(1), jnp.int32(0))

                                cnt = lax.fori_loop(
                                    0, k + 1, cbody, jnp.int32(0))
                                return (m2, r2 + cnt)

                            return lax.cond(m_prev != m, fp_step,
                                            lambda s2: s2, (m_prev, m))

                        _, neg = lax.fori_loop(
                            0, k + 2, fp_body, (jnp.int32(-1), r2))
                        dset_s[k + 1] = neg
                        xn = rows_v[buf, pl.ds(roff + neg, 16)][0]
                        margin = jnp.float32(1.0) + xn - xj
                        return (jnp.int32(k + 1), margin, p)

                    return active

                st = (jnp.int32(0), jnp.float32(-1.0), p)

                def trial_body(k, st):
                    return lax.cond(st[1] < 0.0, make_active(k),
                                    passive, st)

                t, margin, p = lax.fori_loop(0, _MAXT, trial_body, st)
                lval = ltab_v[pl.ds(t - 1, 16)][0]
                acc = acc + jnp.where(margin >= 0.0, lval * margin,
                                      jnp.float32(0.0))
                return (p, acc)

            return row_body

        row_body0 = make_row_body(0)
        row_body1 = make_row_body(1)

        def pair_body(c2, carry):
            c = c2 * 2
            chunk_copy(0, 0, sem0).wait()
            carry = lax.fori_loop(
                0, _RCHUNK, lambda r, cy: row_body0(r, cy, c), carry)

            @pl.when(c2 < _NCHUNK // 2 - 1)
            def _():
                chunk_copy(c + 2, 0, sem0).start()

            chunk_copy(1, 1, sem1).wait()
            carry = lax.fori_loop(
                0, _RCHUNK, lambda r, cy: row_body1(r, cy, c + 1), carry)

            @pl.when(c2 < _NCHUNK // 2 - 1)
            def _():
                chunk_copy(c + 3, 1, sem1).start()

            return carry

        _, acc = lax.fori_loop(0, _NCHUNK // 2, pair_body,
                               (jnp.int32(0), jnp.float32(0.0)))
        out_v[...] = jnp.full((16,), acc, jnp.float32)
        pltpu.sync_copy(out_v, out_hbm)


_sc_call = functools.partial(
    pl.kernel,
    out_type=jax.ShapeDtypeStruct((16,), jnp.float32),
    mesh=plsc.VectorSubcoreMesh(core_axis_name="c", subcore_axis_name="s"),
    compiler_params=pltpu.CompilerParams(needs_layout_passes=False),
    scratch_types=[
        pltpu.VMEM((_NSTREAM + 32,), jnp.int32),
        pltpu.VMEM((_B * 8 + 16,), jnp.int32),
        pltpu.VMEM((2, _RCHUNK * _Y + 16), jnp.float32),
        pltpu.VMEM((_MAXT + 16,), jnp.float32),
        pltpu.VMEM((16,), jnp.float32),
        pltpu.SMEM((_MAXT + 8,), jnp.int32),
        pltpu.SemaphoreType.DMA,
        pltpu.SemaphoreType.DMA,
    ],
)(_sc_body)


def kernel(input, target):
    meta = _extract(target)
    stream = jnp.asarray(_STREAM_HOST)
    ltab = jnp.asarray(_LTAB_HOST)
    out = _sc_call(input.reshape(-1), meta.reshape(-1), stream, ltab)
    return out[:1]


# speedup vs baseline: 81.7970x; 2.5406x over previous
import functools

import numpy as np
import jax
import jax.numpy as jnp
from jax import lax
from jax.experimental import pallas as pl
from jax.experimental.pallas import tpu as pltpu
from jax.experimental.pallas import tpu_sc as plsc

_B = 1024
_Y = 1000
_MAXT = 64
_NSTREAM = 80 * 1024
_RCHUNK = 16
_NCHUNK = _B // _RCHUNK

_STREAM_HOST = (
    np.random.RandomState(0)
    .randint(0, 2**32, size=_NSTREAM, dtype=np.uint32)
    .view(np.int32)
)
_LTAB_HOST = np.log(
    np.floor((float(_Y) - 1.0) / np.arange(1, _MAXT + 1, dtype=np.float64))
).astype(np.float32)


def _max_run(bad):
    idx = np.flatnonzero(np.diff(np.concatenate(([0], bad.view(np.uint8), [0]))))
    return int((idx[1::2] - idx[::2]).max()) if idx.size else 0


_SU = _STREAM_HOST.view(np.uint32)
_NEG_DEPTH = 1 + _max_run((_SU & 1023) > 935)
_POS_DEPTH = 1 + max(_max_run((_SU & 3) == 3), _max_run((_SU & 7) > 4))
assert _NEG_DEPTH <= 16 and _POS_DEPTH <= 16


def _extract_body(t_ref, meta_ref):
    t = t_ref[...]
    iota = lax.broadcasted_iota(jnp.int32, (_B, _Y), 1)
    big = jnp.int32(2048)
    live = jnp.where(t > 0.0, iota, big)
    cols = []
    for _ in range(5):
        mn = jnp.min(live, axis=1)
        cols.append(mn)
        live = jnp.where(live == mn[:, None], big, live)
    npos = jnp.sum((t > 0.0).astype(jnp.int32), axis=1)
    zero = jnp.zeros((_B,), jnp.int32)
    meta_ref[...] = jnp.stack(cols + [npos, zero, zero], axis=1)


_extract = pl.pallas_call(
    _extract_body,
    out_shape=jax.ShapeDtypeStruct((_B, 8), jnp.int32),
)


def _sc_body(inp_hbm, meta_hbm, stream_hbm, ltab_hbm, out_hbm,
             stream_v, meta_v, rows_v, ltab_v, out_v, dset_s, sem0, sem1):
    cid = lax.axis_index("c")
    sid = lax.axis_index("s")

    @pl.when(jnp.logical_and(cid == 0, sid == 0))
    def _serial():
        pltpu.sync_copy(stream_hbm, stream_v.at[pl.ds(0, _NSTREAM)])
        pltpu.sync_copy(meta_hbm, meta_v.at[pl.ds(0, _B * 8)])
        pltpu.sync_copy(ltab_hbm, ltab_v.at[pl.ds(0, _MAXT)])

        csz = _RCHUNK * _Y

        def chunk_copy(c, buf, sem):
            return pltpu.make_async_copy(
                inp_hbm.at[pl.ds(c * csz, csz)],
                rows_v.at[buf, pl.ds(0, csz)], sem)

        chunk_copy(0, 0, sem0).start()
        chunk_copy(1, 1, sem1).start()

        def draw(p, rngv, mask, depth):
            def attempt(k):
                w = stream_v[pl.ds(p + k, 16)][0]
                v = jnp.bitwise_and(w, mask)
                if k == depth - 1:
                    return (v, p + jnp.int32(k + 1))
                return lax.cond(v <= rngv,
                                lambda: (v, p + jnp.int32(k + 1)),
                                lambda: attempt(k + 1))

            return attempt(0)

        def make_row_body(buf):
            def row_body(r, carry, c):
                p, acc = carry
                i = c * _RCHUNK + r
                roff = r * _Y
                npos = meta_v[pl.ds(i * 8 + 5, 16)][0]

                def posdraw(pp):
                    rngv = npos - 1
                    m = rngv
                    m = lax.bitwise_or(m, m >> 1)
                    m = lax.bitwise_or(m, m >> 2)
                    return draw(pp, rngv, m, _POS_DEPTH)

                rr, p = lax.cond(npos > 1, posdraw,
                                 lambda pp: (jnp.int32(0), pp), p)
                j = meta_v[pl.ds(i * 8 + rr, 16)][0]
                xj = rows_v[buf, pl.ds(roff + j, 16)][0]
                dset_s[0] = j

                def passive(st):
                    return st

                def make_active(k):
                    def active(st):
                        _, _, p = st
                        rngv2 = jnp.int32(_Y - 2) - k
                        r2, p = draw(p, rngv2, jnp.int32(1023), _NEG_DEPTH)

                        def fp_body(_, s):
                            m_prev, m = s

                            def fp_step(s2):
                                _, m2 = s2

                                def cbody(q, cnt):
                                    return cnt + jnp.where(
                                        dset_s[q] <= m2,
                                        jnp.int32(1), jnp.int32(0))

                                cnt = lax.fori_loop(
                                    0, k + 1, cbody, jnp.int32(0))
                                return (m2, r2 + cnt)

                            return lax.cond(m_prev != m, fp_step,
                                            lambda s2: s2, (m_prev, m))

                        _, neg = lax.fori_loop(
                            0, k + 2, fp_body, (jnp.int32(-1), r2))
                        dset_s[k + 1] = neg
                        xn = rows_v[buf, pl.ds(roff + neg, 16)][0]
                        margin = jnp.float32(1.0) + xn - xj
                        return (jnp.int32(k + 1), margin, p)

                    return active

                st = (jnp.int32(0), jnp.float32(-1.0), p)

                def trial_body(k, st):
                    return lax.cond(st[1] < 0.0, make_active(k),
                                    passive, st)

                for k in range(4):
                    st = trial_body(k, st)
                st = lax.cond(
                    st[1] < 0.0,
                    lambda s: lax.fori_loop(4, _MAXT, trial_body, s),
                    passive, st)
                t, margin, p = st
                lval = ltab_v[pl.ds(t - 1, 16)][0]
                acc = acc + jnp.where(margin >= 0.0, lval * margin,
                                      jnp.float32(0.0))
                return (p, acc)

            return row_body

        row_body0 = make_row_body(0)
        row_body1 = make_row_body(1)

        def pair_body(c2, carry):
            c = c2 * 2
            chunk_copy(0, 0, sem0).wait()
            carry = lax.fori_loop(
                0, _RCHUNK, lambda r, cy: row_body0(r, cy, c), carry)

            @pl.when(c2 < _NCHUNK // 2 - 1)
            def _():
                chunk_copy(c + 2, 0, sem0).start()

            chunk_copy(1, 1, sem1).wait()
            carry = lax.fori_loop(
                0, _RCHUNK, lambda r, cy: row_body1(r, cy, c + 1), carry)

            @pl.when(c2 < _NCHUNK // 2 - 1)
            def _():
                chunk_copy(c + 3, 1, sem1).start()

            return carry

        _, acc = lax.fori_loop(0, _NCHUNK // 2, pair_body,
                               (jnp.int32(0), jnp.float32(0.0)))
        out_v[...] = jnp.full((16,), acc, jnp.float32)
        pltpu.sync_copy(out_v, out_hbm)


_sc_call = functools.partial(
    pl.kernel,
    out_type=jax.ShapeDtypeStruct((16,), jnp.float32),
    mesh=plsc.VectorSubcoreMesh(core_axis_name="c", subcore_axis_name="s"),
    compiler_params=pltpu.CompilerParams(needs_layout_passes=False),
    scratch_types=[
        pltpu.VMEM((_NSTREAM + 32,), jnp.int32),
        pltpu.VMEM((_B * 8 + 16,), jnp.int32),
        pltpu.VMEM((2, _RCHUNK * _Y + 16), jnp.float32),
        pltpu.VMEM((_MAXT + 16,), jnp.float32),
        pltpu.VMEM((16,), jnp.float32),
        pltpu.SMEM((_MAXT + 8,), jnp.int32),
        pltpu.SemaphoreType.DMA,
        pltpu.SemaphoreType.DMA,
    ],
)(_sc_body)


def kernel(input, target):
    meta = _extract(target)
    stream = jnp.asarray(_STREAM_HOST)
    ltab = jnp.asarray(_LTAB_HOST)
    out = _sc_call(input.reshape(-1), meta.reshape(-1), stream, ltab)
    return out[:1]


# speedup vs baseline: 83.1354x; 1.0164x over previous
import functools

import numpy as np
import jax
import jax.numpy as jnp
from jax import lax
from jax.experimental import pallas as pl
from jax.experimental.pallas import tpu as pltpu
from jax.experimental.pallas import tpu_sc as plsc

_B = 1024
_Y = 1000
_MAXT = 64
_NSTREAM = 80 * 1024
_RCHUNK = 16
_NCHUNK = _B // _RCHUNK

_STREAM_HOST = (
    np.random.RandomState(0)
    .randint(0, 2**32, size=_NSTREAM, dtype=np.uint32)
    .view(np.int32)
)
_LTAB_HOST = np.log(
    np.floor((float(_Y) - 1.0) / np.arange(1, _MAXT + 1, dtype=np.float64))
).astype(np.float32)


def _max_run(bad):
    idx = np.flatnonzero(np.diff(np.concatenate(([0], bad.view(np.uint8), [0]))))
    return int((idx[1::2] - idx[::2]).max()) if idx.size else 0


_SU = _STREAM_HOST.view(np.uint32)
_NEG_DEPTH = 1 + _max_run((_SU & 1023) > 935)
_POS_DEPTH = 1 + max(_max_run((_SU & 3) == 3), _max_run((_SU & 7) > 4))
assert _NEG_DEPTH <= 16 and _POS_DEPTH <= 16


def _extract_body(t_ref, meta_ref):
    t = t_ref[...]
    iota = lax.broadcasted_iota(jnp.int32, (_B, _Y), 1)
    big = jnp.int32(2048)
    live = jnp.where(t > 0.0, iota, big)
    cols = []
    for _ in range(5):
        mn = jnp.min(live, axis=1)
        cols.append(mn)
        live = jnp.where(live == mn[:, None], big, live)
    npos = jnp.sum((t > 0.0).astype(jnp.int32), axis=1)
    zero = jnp.zeros((_B,), jnp.int32)
    meta_ref[...] = jnp.stack(cols + [npos, zero, zero], axis=1)


_extract = pl.pallas_call(
    _extract_body,
    out_shape=jax.ShapeDtypeStruct((_B, 8), jnp.int32),
)


def _sc_body(inp_hbm, meta_hbm, stream_hbm, ltab_hbm, out_hbm,
             stream_v, meta_v, rows_v, ltab_v, out_v, dset_s, sem0, sem1):
    cid = lax.axis_index("c")
    sid = lax.axis_index("s")

    @pl.when(jnp.logical_and(cid == 0, sid == 0))
    def _serial():
        pltpu.sync_copy(stream_hbm, stream_v.at[pl.ds(0, _NSTREAM)])
        pltpu.sync_copy(meta_hbm, meta_v.at[pl.ds(0, _B * 8)])
        pltpu.sync_copy(ltab_hbm, ltab_v.at[pl.ds(0, _MAXT)])

        csz = _RCHUNK * _Y

        def chunk_copy(c, buf, sem):
            return pltpu.make_async_copy(
                inp_hbm.at[pl.ds(c * csz, csz)],
                rows_v.at[buf, pl.ds(0, csz)], sem)

        chunk_copy(0, 0, sem0).start()
        chunk_copy(1, 1, sem1).start()

        def draw(p, rngv, mask, depth):
            def attempt(k):
                w = stream_v[pl.ds(p + k, 16)][0]
                v = jnp.bitwise_and(w, mask)
                if k == depth - 1:
                    return (v, p + jnp.int32(k + 1))
                return lax.cond(v <= rngv,
                                lambda: (v, p + jnp.int32(k + 1)),
                                lambda: attempt(k + 1))

            return attempt(0)

        def vsel(vec, idx):
            idx16 = jnp.full((16,), idx, jnp.int32)
            return lax.gather(
                vec, idx16[:, None],
                lax.GatherDimensionNumbers(offset_dims=(),
                                           collapsed_slice_dims=(0,),
                                           start_index_map=(0,)),
                (1,), mode=lax.GatherScatterMode.PROMISE_IN_BOUNDS)[0]

        _NUNROLL = 4

        def make_row_body(buf):
            def row_body(r, carry, c):
                p, acc = carry
                i = c * _RCHUNK + r
                roff = r * _Y
                meta16 = meta_v[pl.ds(i * 8, 16)]
                npos = meta16[5]

                def posdraw(pp):
                    rngv = npos - 1
                    m = rngv
                    m = lax.bitwise_or(m, m >> 1)
                    m = lax.bitwise_or(m, m >> 2)
                    return draw(pp, rngv, m, _POS_DEPTH)

                rr, p = lax.cond(npos > 1, posdraw,
                                 lambda pp: (jnp.int32(0), pp), p)
                j = vsel(meta16, rr)
                xj = rows_v[buf, pl.ds(roff + j, 16)][0]

                def passive(st):
                    return st

                def make_active_reg(k):
                    def active(st):
                        _, _, p, ns = st
                        rngv2 = jnp.int32(_Y - 2 - k)
                        r2, p = draw(p, rngv2, jnp.int32(1023), _NEG_DEPTH)
                        dels = (j,) + ns[:k]
                        m = r2
                        for _ in range(k + 2):
                            cnt = jnp.int32(0)
                            for d in dels:
                                cnt = cnt + jnp.where(d <= m, jnp.int32(1),
                                                      jnp.int32(0))
                            m = r2 + cnt
                        neg = m
                        xn = rows_v[buf, pl.ds(roff + neg, 16)][0]
                        margin = jnp.float32(1.0) + xn - xj
                        ns = ns[:k] + (neg,) + ns[k + 1:]
                        return (jnp.int32(k + 1), margin, p, ns)

                    return active

                ns0 = (jnp.int32(0),) * _NUNROLL
                st = (jnp.int32(0), jnp.float32(-1.0), p, ns0)
                for k in range(_NUNROLL):
                    st = lax.cond(st[1] < 0.0, make_active_reg(k),
                                  passive, st)

                def tail(st):
                    t0, margin0, p0, ns = st
                    dset_s[0] = j
                    for k in range(_NUNROLL):
                        dset_s[k + 1] = ns[k]

                    def active_dyn(st2):
                        t, _, p2 = st2
                        rngv2 = jnp.int32(_Y - 2) - t
                        r2, p2 = draw(p2, rngv2, jnp.int32(1023),
                                      _NEG_DEPTH)

                        def fp_body(_, s):
                            m_prev, m = s

                            def fp_step(s2):
                                _, m2 = s2

                                def cbody(q, cnt):
                                    return cnt + jnp.where(
                                        dset_s[q] <= m2,
                                        jnp.int32(1), jnp.int32(0))

                                cnt = lax.fori_loop(0, t + 1, cbody,
                                                    jnp.int32(0))
                                return (m2, r2 + cnt)

                            return lax.cond(m_prev != m, fp_step,
                                            lambda s2: s2, (m_prev, m))

                        _, neg = lax.fori_loop(0, t + 2, fp_body,
                                               (jnp.int32(-1), r2))
                        dset_s[t + 1] = neg
                        xn = rows_v[buf, pl.ds(roff + neg, 16)][0]
                        margin = jnp.float32(1.0) + xn - xj
                        return (t + 1, margin, p2)

                    def trial_dyn(k, st2):
                        return lax.cond(st2[1] < 0.0, active_dyn,
                                        passive, st2)

                    t, margin, p1 = lax.fori_loop(
                        _NUNROLL, _MAXT, trial_dyn, (t0, margin0, p0))
                    return (t, margin, p1, ns)

                st = lax.cond(st[1] < 0.0, tail, passive, st)
                t, margin, p = st[0], st[1], st[2]
                lval = ltab_v[pl.ds(t - 1, 16)][0]
                acc = acc + jnp.where(margin >= 0.0, lval * margin,
                                      jnp.float32(0.0))
                return (p, acc)

            return row_body

        row_body0 = make_row_body(0)
        row_body1 = make_row_body(1)

        def pair_body(c2, carry):
            c = c2 * 2
            chunk_copy(0, 0, sem0).wait()
            carry = lax.fori_loop(
                0, _RCHUNK, lambda r, cy: row_body0(r, cy, c), carry)

            @pl.when(c2 < _NCHUNK // 2 - 1)
            def _():
                chunk_copy(c + 2, 0, sem0).start()

            chunk_copy(1, 1, sem1).wait()
            carry = lax.fori_loop(
                0, _RCHUNK, lambda r, cy: row_body1(r, cy, c + 1), carry)

            @pl.when(c2 < _NCHUNK // 2 - 1)
            def _():
                chunk_copy(c + 3, 1, sem1).start()

            return carry

        _, acc = lax.fori_loop(0, _NCHUNK // 2, pair_body,
                               (jnp.int32(0), jnp.float32(0.0)))
        out_v[...] = jnp.full((16,), acc, jnp.float32)
        pltpu.sync_copy(out_v, out_hbm)


_sc_call = functools.partial(
    pl.kernel,
    out_type=jax.ShapeDtypeStruct((16,), jnp.float32),
    mesh=plsc.VectorSubcoreMesh(core_axis_name="c", subcore_axis_name="s"),
    compiler_params=pltpu.CompilerParams(needs_layout_passes=False),
    scratch_types=[
        pltpu.VMEM((_NSTREAM + 32,), jnp.int32),
        pltpu.VMEM((_B * 8 + 16,), jnp.int32),
        pltpu.VMEM((2, _RCHUNK * _Y + 16), jnp.float32),
        pltpu.VMEM((_MAXT + 16,), jnp.float32),
        pltpu.VMEM((16,), jnp.float32),
        pltpu.SMEM((_MAXT + 8,), jnp.int32),
        pltpu.SemaphoreType.DMA,
        pltpu.SemaphoreType.DMA,
    ],
)(_sc_body)


def kernel(input, target):
    meta = _extract(target)
    stream = jnp.asarray(_STREAM_HOST)
    ltab = jnp.asarray(_LTAB_HOST)
    out = _sc_call(input.reshape(-1), meta.reshape(-1), stream, ltab)
    return out[:1]


# speedup vs baseline: 93.0071x; 1.1187x over previous
import functools

import numpy as np
import jax
import jax.numpy as jnp
from jax import lax
from jax.experimental import pallas as pl
from jax.experimental.pallas import tpu as pltpu
from jax.experimental.pallas import tpu_sc as plsc

_B = 1024
_Y = 1000
_MAXT = 64
_NSTREAM = 80 * 1024
_RCHUNK = 16
_NCHUNK = _B // _RCHUNK

_STREAM_HOST = (
    np.random.RandomState(0)
    .randint(0, 2**32, size=_NSTREAM, dtype=np.uint32)
    .view(np.int32)
)
_LTAB_HOST = np.log(
    np.floor((float(_Y) - 1.0) / np.arange(1, _MAXT + 1, dtype=np.float64))
).astype(np.float32)


def _max_run(bad):
    idx = np.flatnonzero(np.diff(np.concatenate(([0], bad.view(np.uint8), [0]))))
    return int((idx[1::2] - idx[::2]).max()) if idx.size else 0


_SU = _STREAM_HOST.view(np.uint32)
_NEG_DEPTH = 1 + _max_run((_SU & 1023) > 935)
_POS_DEPTH = 1 + max(_max_run((_SU & 3) == 3), _max_run((_SU & 7) > 4))
assert _NEG_DEPTH <= 16 and _POS_DEPTH <= 16


def _extract_body(t_ref, meta_ref):
    t = t_ref[...]
    iota = lax.broadcasted_iota(jnp.int32, (_B, _Y), 1)
    big = jnp.int32(2048)
    live = jnp.where(t > 0.0, iota, big)
    cols = []
    for _ in range(5):
        mn = jnp.min(live, axis=1)
        cols.append(mn)
        live = jnp.where(live == mn[:, None], big, live)
    npos = jnp.sum((t > 0.0).astype(jnp.int32), axis=1)
    zero = jnp.zeros((_B,), jnp.int32)
    meta_ref[...] = jnp.stack(cols + [npos, zero, zero], axis=1)


_extract = pl.pallas_call(
    _extract_body,
    out_shape=jax.ShapeDtypeStruct((_B, 8), jnp.int32),
)


def _sc_body(inp_hbm, meta_hbm, stream_hbm, ltab_hbm, out_hbm,
             stream_v, meta_v, rows_v, ltab_v, out_v, dset_s, sem0, sem1):
    cid = lax.axis_index("c")
    sid = lax.axis_index("s")

    @pl.when(jnp.logical_and(cid == 0, sid == 0))
    def _serial():
        pltpu.sync_copy(stream_hbm, stream_v.at[pl.ds(0, _NSTREAM)])
        pltpu.sync_copy(meta_hbm, meta_v.at[pl.ds(0, _B * 8)])
        pltpu.sync_copy(ltab_hbm, ltab_v.at[pl.ds(0, _MAXT)])

        csz = _RCHUNK * _Y

        def chunk_copy(c, buf, sem):
            return pltpu.make_async_copy(
                inp_hbm.at[pl.ds(c * csz, csz)],
                rows_v.at[buf, pl.ds(0, csz)], sem)

        chunk_copy(0, 0, sem0).start()
        chunk_copy(1, 1, sem1).start()

        def vsel(vec, idx):
            idx16 = jnp.full((16,), idx, jnp.int32)
            return lax.gather(
                vec, idx16[:, None],
                lax.GatherDimensionNumbers(offset_dims=(),
                                           collapsed_slice_dims=(0,),
                                           start_index_map=(0,)),
                (1,), mode=lax.GatherScatterMode.PROMISE_IN_BOUNDS)[0]

        def draw(p, rngv, mask, depth):
            del depth
            w16 = stream_v[pl.ds(p, 16)]
            v16 = jnp.bitwise_and(w16, jnp.full((16,), mask, jnp.int32))
            ok = v16 <= rngv
            lane = plsc.all_reduce_ffs(ok)
            if getattr(lane, "ndim", 0):
                lane = lane[0]
            return vsel(v16, lane), p + lane + jnp.int32(1)

        _NUNROLL = 4

        def make_row_body(buf):
            def row_body(r, carry, c):
                p, acc = carry
                i = c * _RCHUNK + r
                roff = r * _Y
                meta16 = meta_v[pl.ds(i * 8, 16)]
                npos = meta16[5]

                rngv = npos - 1
                m = rngv
                m = lax.bitwise_or(m, m >> 1)
                m = lax.bitwise_or(m, m >> 2)
                v, pd = draw(p, rngv, m, _POS_DEPTH)
                many = npos > 1
                rr = jnp.where(many, v, jnp.int32(0))
                p = jnp.where(many, pd, p)
                j = vsel(meta16, rr)
                xj = rows_v[buf, pl.ds(roff + j, 16)][0]

                r2, p = draw(p, jnp.int32(_Y - 2), jnp.int32(1023),
                             _NEG_DEPTH)
                neg0 = r2 + jnp.where(j <= r2, jnp.int32(1), jnp.int32(0))
                xn0 = rows_v[buf, pl.ds(roff + neg0, 16)][0]
                margin = jnp.float32(1.0) + xn0 - xj

                def passive(st):
                    return st

                def make_active_reg(k):
                    def active(st):
                        _, _, p, ns = st
                        rngv2 = jnp.int32(_Y - 2 - k)
                        r2, p = draw(p, rngv2, jnp.int32(1023), _NEG_DEPTH)
                        dels = (j,) + ns[:k]
                        m = r2
                        for _ in range(k + 2):
                            cnt = jnp.int32(0)
                            for d in dels:
                                cnt = cnt + jnp.where(d <= m, jnp.int32(1),
                                                      jnp.int32(0))
                            m = r2 + cnt
                        neg = m
                        xn = rows_v[buf, pl.ds(roff + neg, 16)][0]
                        margin = jnp.float32(1.0) + xn - xj
                        ns = ns[:k] + (neg,) + ns[k + 1:]
                        return (jnp.int32(k + 1), margin, p, ns)

                    return active

                ns0 = (neg0,) + (jnp.int32(0),) * (_NUNROLL - 1)
                st = (jnp.int32(1), margin, p, ns0)
                for k in range(1, _NUNROLL):
                    st = lax.cond(st[1] < 0.0, make_active_reg(k),
                                  passive, st)

                def tail(st):
                    t0, margin0, p0, ns = st
                    dset_s[0] = j
                    for k in range(_NUNROLL):
                        dset_s[k + 1] = ns[k]

                    def active_dyn(st2):
                        t, _, p2 = st2
                        rngv2 = jnp.int32(_Y - 2) - t
                        r2, p2 = draw(p2, rngv2, jnp.int32(1023),
                                      _NEG_DEPTH)

                        def fp_body(_, s):
                            m_prev, m = s

                            def fp_step(s2):
                                _, m2 = s2

                                def cbody(q, cnt):
                                    return cnt + jnp.where(
                                        dset_s[q] <= m2,
                                        jnp.int32(1), jnp.int32(0))

                                cnt = lax.fori_loop(0, t + 1, cbody,
                                                    jnp.int32(0))
                                return (m2, r2 + cnt)

                            return lax.cond(m_prev != m, fp_step,
                                            lambda s2: s2, (m_prev, m))

                        _, neg = lax.fori_loop(0, t + 2, fp_body,
                                               (jnp.int32(-1), r2))
                        dset_s[t + 1] = neg
                        xn = rows_v[buf, pl.ds(roff + neg, 16)][0]
                        margin = jnp.float32(1.0) + xn - xj
                        return (t + 1, margin, p2)

                    def trial_dyn(k, st2):
                        return lax.cond(st2[1] < 0.0, active_dyn,
                                        passive, st2)

                    t, margin, p1 = lax.fori_loop(
                        _NUNROLL, _MAXT, trial_dyn, (t0, margin0, p0))
                    return (t, margin, p1, ns)

                st = lax.cond(st[1] < 0.0, tail, passive, st)
                t, margin, p = st[0], st[1], st[2]
                lval = ltab_v[pl.ds(t - 1, 16)][0]
                acc = acc + jnp.where(margin >= 0.0, lval * margin,
                                      jnp.float32(0.0))
                return (p, acc)

            return row_body

        row_body0 = make_row_body(0)
        row_body1 = make_row_body(1)

        def pair_body(c2, carry):
            c = c2 * 2
            chunk_copy(0, 0, sem0).wait()
            carry = lax.fori_loop(
                0, _RCHUNK, lambda r, cy: row_body0(r, cy, c), carry)

            @pl.when(c2 < _NCHUNK // 2 - 1)
            def _():
                chunk_copy(c + 2, 0, sem0).start()

            chunk_copy(1, 1, sem1).wait()
            carry = lax.fori_loop(
                0, _RCHUNK, lambda r, cy: row_body1(r, cy, c + 1), carry)

            @pl.when(c2 < _NCHUNK // 2 - 1)
            def _():
                chunk_copy(c + 3, 1, sem1).start()

            return carry

        _, acc = lax.fori_loop(0, _NCHUNK // 2, pair_body,
                               (jnp.int32(0), jnp.float32(0.0)))
        out_v[...] = jnp.full((16,), acc, jnp.float32)
        pltpu.sync_copy(out_v, out_hbm)


_sc_call = functools.partial(
    pl.kernel,
    out_type=jax.ShapeDtypeStruct((16,), jnp.float32),
    mesh=plsc.VectorSubcoreMesh(core_axis_name="c", subcore_axis_name="s"),
    compiler_params=pltpu.CompilerParams(needs_layout_passes=False),
    scratch_types=[
        pltpu.VMEM((_NSTREAM + 32,), jnp.int32),
        pltpu.VMEM((_B * 8 + 16,), jnp.int32),
        pltpu.VMEM((2, _RCHUNK * _Y + 16), jnp.float32),
        pltpu.VMEM((_MAXT + 16,), jnp.float32),
        pltpu.VMEM((16,), jnp.float32),
        pltpu.SMEM((_MAXT + 8,), jnp.int32),
        pltpu.SemaphoreType.DMA,
        pltpu.SemaphoreType.DMA,
    ],
)(_sc_body)


def kernel(input, target):
    meta = _extract(target)
    stream = jnp.asarray(_STREAM_HOST)
    ltab = jnp.asarray(_LTAB_HOST)
    out = _sc_call(input.reshape(-1), meta.reshape(-1), stream, ltab)
    return out[:1]


# speedup vs baseline: 117.3057x; 1.2613x over previous
import functools

import numpy as np
import jax
import jax.numpy as jnp
from jax import lax
from jax.experimental import pallas as pl
from jax.experimental.pallas import tpu as pltpu
from jax.experimental.pallas import tpu_sc as plsc

_B = 1024
_Y = 1000
_MAXT = 64
_NSTREAM = 80 * 1024
_RCHUNK = 16
_NCHUNK = _B // _RCHUNK

_STREAM_HOST = (
    np.random.RandomState(0)
    .randint(0, 2**32, size=_NSTREAM, dtype=np.uint32)
    .view(np.int32)
)
_LTAB_HOST = np.log(
    np.floor((float(_Y) - 1.0) / np.arange(1, _MAXT + 1, dtype=np.float64))
).astype(np.float32)


def _max_run(bad):
    idx = np.flatnonzero(np.diff(np.concatenate(([0], bad.view(np.uint8), [0]))))
    return int((idx[1::2] - idx[::2]).max()) if idx.size else 0


_SU = _STREAM_HOST.view(np.uint32)
_NEG_DEPTH = 1 + _max_run((_SU & 1023) > 935)
_POS_DEPTH = 1 + max(_max_run((_SU & 3) == 3), _max_run((_SU & 7) > 4))
assert _NEG_DEPTH <= 16 and _POS_DEPTH <= 16


def _extract_body(t_ref, meta_ref):
    t = t_ref[...]
    iota = lax.broadcasted_iota(jnp.int32, (_B, _Y), 1)
    big = jnp.int32(2048)
    live = jnp.where(t > 0.0, iota, big)
    cols = []
    for _ in range(5):
        mn = jnp.min(live, axis=1)
        cols.append(mn)
        live = jnp.where(live == mn[:, None], big, live)
    npos = jnp.sum((t > 0.0).astype(jnp.int32), axis=1)
    zero = jnp.zeros((_B,), jnp.int32)
    meta_ref[...] = jnp.stack(cols + [npos, zero, zero], axis=1)


_extract = pl.pallas_call(
    _extract_body,
    out_shape=jax.ShapeDtypeStruct((_B, 8), jnp.int32),
)


def _sc_body(inp_hbm, meta_hbm, stream_hbm, ltab_hbm, out_hbm,
             stream_v, meta_v, rows_v, ltab_v, out_v, dset_s, sem0, sem1):
    cid = lax.axis_index("c")
    sid = lax.axis_index("s")

    @pl.when(jnp.logical_and(cid == 0, sid == 0))
    def _serial():
        pltpu.sync_copy(stream_hbm, stream_v.at[pl.ds(0, _NSTREAM)])
        pltpu.sync_copy(meta_hbm, meta_v.at[pl.ds(0, _B * 8)])
        pltpu.sync_copy(ltab_hbm, ltab_v.at[pl.ds(0, _MAXT)])

        csz = _RCHUNK * _Y

        def chunk_copy(c, buf, sem):
            return pltpu.make_async_copy(
                inp_hbm.at[pl.ds(c * csz, csz)],
                rows_v.at[buf, pl.ds(0, csz)], sem)

        chunk_copy(0, 0, sem0).start()
        chunk_copy(1, 1, sem1).start()

        iota16 = lax.iota(jnp.int32, 16)
        ones_i = jnp.full((16,), 1, jnp.int32)
        zero_i = jnp.zeros((16,), jnp.int32)

        def splat(x, dtype=jnp.int32):
            return jnp.full((16,), x, dtype)

        def vpick(vec, lanev):
            return lax.gather(
                vec, lanev[:, None],
                lax.GatherDimensionNumbers(offset_dims=(),
                                           collapsed_slice_dims=(0,),
                                           start_index_map=(0,)),
                (1,), mode=lax.GatherScatterMode.PROMISE_IN_BOUNDS)

        def draw_vec(pv, rngv, maskv):
            w16 = plsc.load_gather(stream_v, [pv + iota16])
            v16 = jnp.bitwise_and(w16, maskv)
            ok = v16 <= rngv
            lane = plsc.all_reduce_ffs(ok)
            return vpick(v16, lane), pv + lane + ones_i

        def draw_s(p, rngv, mask):
            w16 = stream_v[pl.ds(p, 16)]
            v16 = jnp.bitwise_and(w16, jnp.full((16,), mask, jnp.int32))
            lane = plsc.all_reduce_ffs(v16 <= rngv)[0]
            v = vpick(v16, splat(lane))[0]
            return v, p + lane + jnp.int32(1)

        _NUNROLL = 4

        def make_row_body(buf):
            bufv = splat(buf)

            def row_body(r, carry, c):
                pv, accv = carry
                i = c * _RCHUNK + r
                roffv = splat(r * _Y)
                meta16 = plsc.load_gather(meta_v, [splat(i * 8) + iota16])
                nposv = vpick(meta16, splat(5))

                rngv = nposv - ones_i
                mv = jnp.bitwise_or(rngv, rngv >> 1)
                mv = jnp.bitwise_or(mv, mv >> 2)
                v, pd = draw_vec(pv, rngv, mv)
                many = nposv > ones_i
                rrv = jnp.where(many, v, zero_i)
                pv = jnp.where(many, pd, pv)
                jv = vpick(meta16, rrv)
                xjv = plsc.load_gather(rows_v, [bufv, roffv + jv])

                r2, pv = draw_vec(pv, splat(_Y - 2), splat(1023))
                neg0 = r2 + jnp.where(jv <= r2, ones_i, zero_i)
                xn0 = plsc.load_gather(rows_v, [bufv, roffv + neg0])
                marginv = jnp.float32(1.0) + xn0 - xjv
                tv = ones_i
                ns = [neg0, zero_i, zero_i, zero_i]

                for k in range(1, _NUNROLL):
                    active = marginv < 0.0
                    r2, pnew = draw_vec(pv, splat(_Y - 2 - k), splat(1023))
                    dels = [jv] + ns[:k]
                    m = r2
                    for _ in range(k + 2):
                        cnt = zero_i
                        for d in dels:
                            cnt = cnt + jnp.where(d <= m, ones_i, zero_i)
                        m = r2 + cnt
                    xnk = plsc.load_gather(rows_v, [bufv, roffv + m])
                    mg = jnp.float32(1.0) + xnk - xjv
                    ns[k] = jnp.where(active, m, zero_i)
                    marginv = jnp.where(active, mg, marginv)
                    pv = jnp.where(active, pnew, pv)
                    tv = jnp.where(active, splat(k + 1), tv)

                def tail(op):
                    pv_, tv_, mv_ = op
                    p0 = pv_[0]
                    margin0 = mv_[0]
                    xj_s = xjv[0]
                    dset_s[0] = jv[0]
                    for k in range(_NUNROLL):
                        dset_s[k + 1] = ns[k][0]
                    return _tail_run(p0, margin0, xj_s, op)

                def _tail_run(p0, margin0, xj_s, op):
                    def active_dyn(st2):
                        t, _, p2 = st2
                        rngv2 = jnp.int32(_Y - 2) - t
                        r2s, p2 = draw_s(p2, rngv2, jnp.int32(1023))

                        def fp_body(_, s):
                            m_prev, m2 = s

                            def fp_step(s2):
                                _, m3 = s2

                                def cbody(q, cnt):
                                    return cnt + jnp.where(
                                        dset_s[q] <= m3,
                                        jnp.int32(1), jnp.int32(0))

                                cnt = lax.fori_loop(0, t + 1, cbody,
                                                    jnp.int32(0))
                                return (m3, r2s + cnt)

                            return lax.cond(m_prev != m2, fp_step,
                                            lambda s2: s2, (m_prev, m2))

                        _, neg = lax.fori_loop(0, t + 2, fp_body,
                                               (jnp.int32(-1), r2s))
                        dset_s[t + 1] = neg
                        xn_s = rows_v[buf, pl.ds(r * _Y + neg, 16)][0]
                        margin = jnp.float32(1.0) + xn_s - xj_s
                        return (t + 1, margin, p2)

                    def trial_dyn(k2, st2):
                        return lax.cond(st2[1] < 0.0, active_dyn,
                                        lambda s: s, st2)

                    t1, mg1, p1 = lax.fori_loop(
                        _NUNROLL, _MAXT, trial_dyn,
                        (jnp.int32(_NUNROLL), margin0, p0))
                    return (splat(p1), splat(t1),
                            jnp.full((16,), mg1, jnp.float32))

                pv, tv, marginv = lax.cond(marginv[0] < 0.0, tail,
                                           lambda op: op,
                                           (pv, tv, marginv))

                lval = plsc.load_gather(ltab_v, [tv - ones_i])
                accv = accv + jnp.where(marginv >= 0.0, lval * marginv,
                                        jnp.float32(0.0))
                return (pv, accv)

            return row_body

        row_body0 = make_row_body(0)
        row_body1 = make_row_body(1)

        def pair_body(c2, carry):
            c = c2 * 2
            chunk_copy(0, 0, sem0).wait()
            carry = lax.fori_loop(
                0, _RCHUNK, lambda r, cy: row_body0(r, cy, c), carry)

            @pl.when(c2 < _NCHUNK // 2 - 1)
            def _():
                chunk_copy(c + 2, 0, sem0).start()

            chunk_copy(1, 1, sem1).wait()
            carry = lax.fori_loop(
                0, _RCHUNK, lambda r, cy: row_body1(r, cy, c + 1), carry)

            @pl.when(c2 < _NCHUNK // 2 - 1)
            def _():
                chunk_copy(c + 3, 1, sem1).start()

            return carry

        _, acc = lax.fori_loop(
            0, _NCHUNK // 2, pair_body,
            (jnp.zeros((16,), jnp.int32), jnp.zeros((16,), jnp.float32)))
        out_v[...] = acc
        pltpu.sync_copy(out_v, out_hbm)


_sc_call = functools.partial(
    pl.kernel,
    out_type=jax.ShapeDtypeStruct((16,), jnp.float32),
    mesh=plsc.VectorSubcoreMesh(core_axis_name="c", subcore_axis_name="s"),
    compiler_params=pltpu.CompilerParams(needs_layout_passes=False),
    scratch_types=[
        pltpu.VMEM((_NSTREAM + 32,), jnp.int32),
        pltpu.VMEM((_B * 8 + 16,), jnp.int32),
        pltpu.VMEM((2, _RCHUNK * _Y + 16), jnp.float32),
        pltpu.VMEM((_MAXT + 16,), jnp.float32),
        pltpu.VMEM((16,), jnp.float32),
        pltpu.SMEM((_MAXT + 8,), jnp.int32),
        pltpu.SemaphoreType.DMA,
        pltpu.SemaphoreType.DMA,
    ],
)(_sc_body)


def kernel(input, target):
    meta = _extract(target)
    stream = jnp.asarray(_STREAM_HOST)
    ltab = jnp.asarray(_LTAB_HOST)
    out = _sc_call(input.reshape(-1), meta.reshape(-1), stream, ltab)
    return out[:1]
